# baseline (speedup 1.0000x reference)
import jax
import jax.numpy as jnp
from jax.experimental import pallas as pl
from jax.experimental.pallas import tpu as pltpu

B = 32
N = 2048 * 1024
TOPK = 16
CHUNK = 65536
GRID = N // CHUNK


def _body(o_ref, l_ref, out_ref, acc_ref):
    step = pl.program_id(0)

    @pl.when(step == 0)
    def _init():
        acc_ref[...] = jnp.zeros_like(acc_ref)

    d = o_ref[...] - l_ref[...]
    sq = d * d
    acc_ref[...] += jnp.sum(sq.reshape(B, CHUNK // 128, 128), axis=1)

    @pl.when(step == GRID - 1)
    def _finalize():
        vals0 = jnp.sum(acc_ref[...], axis=1, keepdims=True) * (1.0 / N)
        ii = jax.lax.broadcasted_iota(jnp.int32, (B, 1), 0)
        jk = jax.lax.broadcasted_iota(jnp.int32, (1, TOPK), 1)

        def _extract(k, carry):
            vals, outr = carry
            m = jnp.max(vals)
            outr = jnp.where(jk == k, m, outr)
            first = jnp.min(jnp.where(vals == m, ii, 2 * B))
            vals = jnp.where(ii == first, -jnp.inf, vals)
            return vals, outr

        _, outr = jax.lax.fori_loop(0, TOPK, _extract,
                                    (vals0, jnp.zeros((1, TOPK), jnp.float32)))
        out_ref[...] = outr


def kernel(output, label):
    o2 = output.reshape(B, N)
    l2 = label.reshape(B, N)
    out = pl.pallas_call(
        _body,
        grid=(GRID,),
        in_specs=[pl.BlockSpec((B, CHUNK), lambda i: (0, i)),
                  pl.BlockSpec((B, CHUNK), lambda i: (0, i))],
        out_specs=pl.BlockSpec((1, TOPK), lambda i: (0, 0)),
        out_shape=jax.ShapeDtypeStruct((1, TOPK), jnp.float32),
        scratch_shapes=[pltpu.VMEM((B, 128), jnp.float32)],
    )(o2, l2)
    return out[0]


# speedup vs baseline: 3.4370x; 3.4370x over previous
import jax
import jax.numpy as jnp
from jax.experimental import pallas as pl
from jax.experimental.pallas import tpu as pltpu

B = 32
ROWS = 2048
COLS = 1024
N = ROWS * COLS
TOPK = 16
BR = 1024
STEPS_PER_SAMPLE = ROWS // BR
GRID = B * STEPS_PER_SAMPLE


def _body(o_ref, l_ref, out_ref, acc_ref):
    step = pl.program_id(0)
    sample = step // STEPS_PER_SAMPLE

    d = o_ref[...] - l_ref[...]
    s = jnp.sum(d * d)

    @pl.when(step % STEPS_PER_SAMPLE == 0)
    def _first():
        acc_ref[sample] = s

    @pl.when(step % STEPS_PER_SAMPLE != 0)
    def _rest():
        acc_ref[sample] += s

    @pl.when(step == GRID - 1)
    def _finalize():
        ii = jax.lax.broadcasted_iota(jnp.int32, (B, 1), 0)
        jk = jax.lax.broadcasted_iota(jnp.int32, (1, TOPK), 1)

        def _build(i, vals):
            return jnp.where(ii == i, acc_ref[i], vals)

        vals0 = jax.lax.fori_loop(0, B, _build,
                                  jnp.zeros((B, 1), jnp.float32)) * (1.0 / N)

        def _extract(k, carry):
            vals, outr = carry
            m = jnp.max(vals)
            outr = jnp.where(jk == k, m, outr)
            first = jnp.min(jnp.where(vals == m, ii, 2 * B))
            vals = jnp.where(ii == first, -jnp.inf, vals)
            return vals, outr

        _, outr = jax.lax.fori_loop(0, TOPK, _extract,
                                    (vals0, jnp.zeros((1, TOPK), jnp.float32)))
        out_ref[...] = outr


def kernel(output, label):
    o2 = output.reshape(B * ROWS, COLS)
    l2 = label.reshape(B * ROWS, COLS)
    out = pl.pallas_call(
        _body,
        grid=(GRID,),
        in_specs=[pl.BlockSpec((BR, COLS), lambda i: (i, 0)),
                  pl.BlockSpec((BR, COLS), lambda i: (i, 0))],
        out_specs=pl.BlockSpec((1, TOPK), lambda i: (0, 0)),
        out_shape=jax.ShapeDtypeStruct((1, TOPK), jnp.float32),
        scratch_shapes=[pltpu.SMEM((B,), jnp.float32)],
    )(o2, l2)
    return out[0]
